# Initial kernel scaffold; baseline (speedup 1.0000x reference)
import jax
import jax.numpy as jnp
from jax.experimental import pallas as pl


def kernel(feats, edge_dict, ite, Wkk, bkk, Wk1, bk1, W1, b1, W2, b2, Wfc, bfc):
    raise NotImplementedError("write your pallas kernel here")


# speedup vs baseline: 3.3100x; 3.3100x over previous
import functools

import jax
import jax.numpy as jnp
from jax import lax
from jax.experimental import pallas as pl
from jax.experimental.pallas import tpu as pltpu
from jax.experimental.pallas import tpu_sc as plsc

N = 10000
D = 128
KN = 16
NP = 10240
RB = 256
CB = 2048
NCHUNK = NP // CB
BN = 512
NEG = -3.0e38

NW = 32
PER_W = KN * NP // NW
GCHUNK = 128


def _normalize_body(f_ref, x_ref):
    f = f_ref[...]
    n = jnp.sqrt(jnp.sum(f * f, axis=1, keepdims=True))
    x_ref[...] = f / jnp.maximum(n, 1e-12)


def _topk_body(xr_ref, x_ref, idx_ref, cv_ref, ci_ref):
    xr = xr_ref[...]
    cv_ref[...] = jnp.full((RB, 2 * KN), NEG, jnp.float32)
    ci_ref[...] = jnp.zeros((RB, 2 * KN), jnp.int32)
    iota_c = lax.broadcasted_iota(jnp.int32, (RB, CB), 1)
    iota_m = lax.broadcasted_iota(jnp.int32, (RB, 2 * KN), 1)
    for j in range(NCHUNK):
        xc = x_ref[pl.ds(j * CB, CB), :]
        sims = lax.dot_general(xr, xc, (((1,), (1,)), ((), ())),
                               preferred_element_type=jnp.float32)
        if (j + 1) * CB > N:
            sims = jnp.where(j * CB + iota_c < N, sims, NEG)
        for t in range(KN):
            m = jnp.max(sims, axis=1, keepdims=True)
            am = jnp.min(jnp.where(sims == m, iota_c, CB),
                         axis=1, keepdims=True)
            cv_ref[:, KN + t:KN + t + 1] = m
            ci_ref[:, KN + t:KN + t + 1] = am + j * CB
            sims = jnp.where(iota_c == am, NEG, sims)
        mv = cv_ref[...]
        mi = ci_ref[...]
        for t in range(KN):
            m = jnp.max(mv, axis=1, keepdims=True)
            am = jnp.min(jnp.where(mv == m, iota_m, 2 * KN),
                         axis=1, keepdims=True)
            sel = iota_m == am
            cv_ref[:, t:t + 1] = m
            ci_ref[:, t:t + 1] = jnp.sum(jnp.where(sel, mi, 0),
                                         axis=1, keepdims=True)
            mv = jnp.where(sel, NEG, mv)
    idx_ref[...] = ci_ref[:, 0:KN]


def _vconv_body(near_ref, wkk_ref, bkk_ref, wk1_ref, bk1_ref, wfc_ref,
                bfc_ref, out_ref):
    w = jnp.zeros((BN, KN), jnp.float32)
    for a in range(KN):
        na = near_ref[a]
        mult = lax.dot_general(na, wkk_ref[a], (((1,), (1,)), ((), ())),
                               preferred_element_type=jnp.float32,
                               precision=lax.Precision.HIGHEST)
        mult = mult + bkk_ref[a]
        m = jnp.max(mult, axis=1, keepdims=True)
        e = jnp.exp(mult - m)
        soft = e / jnp.sum(e, axis=1, keepdims=True)
        w = w + wk1_ref[:, a:a + 1] * soft
    pooled = w[:, 0:1] * near_ref[0]
    for b in range(1, KN):
        pooled = pooled + w[:, b:b + 1] * near_ref[b]
    pooled = pooled + bk1_ref[:, 0:1]
    out = lax.dot_general(pooled, wfc_ref[...], (((1,), (1,)), ((), ())),
                          preferred_element_type=jnp.float32,
                          precision=lax.Precision.HIGHEST)
    out_ref[...] = out + bfc_ref[...]


def _make_gather():
    mesh = plsc.VectorSubcoreMesh(core_axis_name="c", subcore_axis_name="s")

    @functools.partial(
        pl.kernel, mesh=mesh,
        out_type=jax.ShapeDtypeStruct((KN * NP, D), jnp.float32),
        scratch_types=[
            pltpu.VMEM((GCHUNK,), jnp.int32),
            pltpu.VMEM((GCHUNK, D), jnp.float32),
            pltpu.SemaphoreType.DMA,
        ],
    )
    def gk(idx_hbm, table_hbm, out_hbm, idx_v, rows_v, sem):
        wid = lax.axis_index("s") * 2 + lax.axis_index("c")
        base = wid * PER_W

        def body(t, carry):
            off = base + t * GCHUNK
            pltpu.sync_copy(idx_hbm.at[pl.ds(off, GCHUNK)], idx_v)
            pltpu.async_copy(table_hbm.at[idx_v], rows_v, sem).wait()
            pltpu.sync_copy(rows_v, out_hbm.at[pl.ds(off, GCHUNK)])
            return carry

        lax.fori_loop(0, PER_W // GCHUNK, body, 0)

    return gk


def kernel(feats, edge_dict, ite, Wkk, bkk, Wk1, bk1, W1, b1, W2, b2, Wfc,
           bfc):
    feats_p = jnp.pad(feats, ((0, NP - N), (0, 0)))

    x = pl.pallas_call(
        _normalize_body,
        out_shape=jax.ShapeDtypeStruct((NP, D), jnp.float32),
    )(feats_p)

    idx = pl.pallas_call(
        _topk_body,
        grid=(NP // RB,),
        in_specs=[
            pl.BlockSpec((RB, D), lambda i: (i, 0)),
            pl.BlockSpec((NP, D), lambda i: (0, 0)),
        ],
        out_specs=pl.BlockSpec((RB, KN), lambda i: (i, 0)),
        out_shape=jax.ShapeDtypeStruct((NP, KN), jnp.int32),
        scratch_shapes=[
            pltpu.VMEM((RB, 2 * KN), jnp.float32),
            pltpu.VMEM((RB, 2 * KN), jnp.int32),
        ],
    )(x, x)

    idx_t = idx.T.reshape(-1)
    nearest = _make_gather()(idx_t, feats)
    near3 = nearest.reshape(KN, NP, D)

    out_p = pl.pallas_call(
        _vconv_body,
        grid=(NP // BN,),
        in_specs=[
            pl.BlockSpec((KN, BN, D), lambda i: (0, i, 0)),
            pl.BlockSpec((KN, KN, D), lambda i: (0, 0, 0)),
            pl.BlockSpec((KN, 1, KN), lambda i: (0, 0, 0)),
            pl.BlockSpec((1, KN), lambda i: (0, 0)),
            pl.BlockSpec((1, 1), lambda i: (0, 0)),
            pl.BlockSpec((D, D), lambda i: (0, 0)),
            pl.BlockSpec((1, D), lambda i: (0, 0)),
        ],
        out_specs=pl.BlockSpec((BN, D), lambda i: (i, 0)),
        out_shape=jax.ShapeDtypeStruct((NP, D), jnp.float32),
    )(near3, Wkk.reshape(KN, KN, D), bkk.reshape(KN, 1, KN),
      Wk1.reshape(1, KN), bk1.reshape(1, 1), Wfc, bfc.reshape(1, D))

    return out_p[:N]


# speedup vs baseline: 4.7006x; 1.4201x over previous
import functools

import jax
import jax.numpy as jnp
from jax import lax
from jax.experimental import pallas as pl
from jax.experimental.pallas import tpu as pltpu
from jax.experimental.pallas import tpu_sc as plsc

N = 10000
D = 128
KN = 16
NP = 10240
RB = 256
CB = 2048
NCHUNK = NP // CB
BN = 512
NEG = -3.0e38

NW = 32
PER_W = KN * NP // NW
GCHUNK = 128


def _normalize_body(f_ref, x_ref):
    f = f_ref[...]
    n = jnp.sqrt(jnp.sum(f * f, axis=1, keepdims=True))
    x_ref[...] = f / jnp.maximum(n, 1e-12)


def _topk_body(xr_ref, x_ref, idx_ref):
    xr = xr_ref[...]
    sims = lax.dot_general(xr, x_ref[...], (((1,), (1,)), ((), ())),
                           preferred_element_type=jnp.float32)
    iota = lax.broadcasted_iota(jnp.int32, (RB, NP), 1)
    sims = jnp.where(iota < N, sims, NEG)
    for t in range(KN):
        am = jnp.argmax(sims, axis=1)[:, None]
        idx_ref[:, t:t + 1] = am.astype(jnp.int32)
        if t + 1 < KN:
            sims = jnp.where(iota == am, NEG, sims)


def _vconv_body(near_ref, wkk_ref, bkk_ref, wk1_ref, bk1_ref, wfc_ref,
                bfc_ref, out_ref):
    w = jnp.zeros((BN, KN), jnp.float32)
    for a in range(KN):
        na = near_ref[a]
        mult = lax.dot_general(na, wkk_ref[a], (((1,), (1,)), ((), ())),
                               preferred_element_type=jnp.float32,
                               precision=lax.Precision.HIGHEST)
        mult = mult + bkk_ref[a]
        m = jnp.max(mult, axis=1, keepdims=True)
        e = jnp.exp(mult - m)
        soft = e / jnp.sum(e, axis=1, keepdims=True)
        w = w + wk1_ref[:, a:a + 1] * soft
    pooled = w[:, 0:1] * near_ref[0]
    for b in range(1, KN):
        pooled = pooled + w[:, b:b + 1] * near_ref[b]
    pooled = pooled + bk1_ref[:, 0:1]
    out = lax.dot_general(pooled, wfc_ref[...], (((1,), (1,)), ((), ())),
                          preferred_element_type=jnp.float32,
                          precision=lax.Precision.HIGHEST)
    out_ref[...] = out + bfc_ref[...]


def _make_gather():
    mesh = plsc.VectorSubcoreMesh(core_axis_name="c", subcore_axis_name="s")

    @functools.partial(
        pl.kernel, mesh=mesh,
        out_type=jax.ShapeDtypeStruct((KN * NP, D), jnp.float32),
        scratch_types=[
            pltpu.VMEM((GCHUNK,), jnp.int32),
            pltpu.VMEM((GCHUNK, D), jnp.float32),
            pltpu.SemaphoreType.DMA,
        ],
    )
    def gk(idx_hbm, table_hbm, out_hbm, idx_v, rows_v, sem):
        wid = lax.axis_index("s") * 2 + lax.axis_index("c")
        base = wid * PER_W

        def body(t, carry):
            off = base + t * GCHUNK
            pltpu.sync_copy(idx_hbm.at[pl.ds(off, GCHUNK)], idx_v)
            pltpu.async_copy(table_hbm.at[idx_v], rows_v, sem).wait()
            pltpu.sync_copy(rows_v, out_hbm.at[pl.ds(off, GCHUNK)])
            return carry

        lax.fori_loop(0, PER_W // GCHUNK, body, 0)

    return gk


def kernel(feats, edge_dict, ite, Wkk, bkk, Wk1, bk1, W1, b1, W2, b2, Wfc,
           bfc):
    feats_p = jnp.pad(feats, ((0, NP - N), (0, 0)))

    x = pl.pallas_call(
        _normalize_body,
        out_shape=jax.ShapeDtypeStruct((NP, D), jnp.float32),
    )(feats_p)

    idx = pl.pallas_call(
        _topk_body,
        grid=(NP // RB,),
        in_specs=[
            pl.BlockSpec((RB, D), lambda i: (i, 0)),
            pl.BlockSpec((NP, D), lambda i: (0, 0)),
        ],
        out_specs=pl.BlockSpec((RB, KN), lambda i: (i, 0)),
        out_shape=jax.ShapeDtypeStruct((NP, KN), jnp.int32),
    )(x, x)

    idx_t = idx.T.reshape(-1)
    nearest = _make_gather()(idx_t, feats)
    near3 = nearest.reshape(KN, NP, D)

    out_p = pl.pallas_call(
        _vconv_body,
        grid=(NP // BN,),
        in_specs=[
            pl.BlockSpec((KN, BN, D), lambda i: (0, i, 0)),
            pl.BlockSpec((KN, KN, D), lambda i: (0, 0, 0)),
            pl.BlockSpec((KN, 1, KN), lambda i: (0, 0, 0)),
            pl.BlockSpec((1, KN), lambda i: (0, 0)),
            pl.BlockSpec((1, 1), lambda i: (0, 0)),
            pl.BlockSpec((D, D), lambda i: (0, 0)),
            pl.BlockSpec((1, D), lambda i: (0, 0)),
        ],
        out_specs=pl.BlockSpec((BN, D), lambda i: (i, 0)),
        out_shape=jax.ShapeDtypeStruct((NP, D), jnp.float32),
    )(near3, Wkk.reshape(KN, KN, D), bkk.reshape(KN, 1, KN),
      Wk1.reshape(1, KN), bk1.reshape(1, 1), Wfc, bfc.reshape(1, D))

    return out_p[:N]


# speedup vs baseline: 4.7967x; 1.0205x over previous
import functools

import jax
import jax.numpy as jnp
from jax import lax
from jax.experimental import pallas as pl
from jax.experimental.pallas import tpu as pltpu
from jax.experimental.pallas import tpu_sc as plsc

N = 10000
D = 128
KN = 16
NP = 10240
RB = 256
CB = 2048
NCHUNK = NP // CB
BN = 512
NEG = -3.0e38

NW = 32
PER_W = KN * NP // NW
GCHUNK = 128


def _normalize_body(f_ref, x_ref):
    f = f_ref[...]
    n = jnp.sqrt(jnp.sum(f * f, axis=1, keepdims=True))
    x_ref[...] = f / jnp.maximum(n, 1e-12)


def _topk_body(xr_ref, x_ref, idx_ref):
    xr = xr_ref[...]
    sims = lax.dot_general(xr, x_ref[...], (((1,), (1,)), ((), ())),
                           preferred_element_type=jnp.float32)
    iota = lax.broadcasted_iota(jnp.int32, (RB, NP), 1)
    sims = jnp.where(iota < N, sims, NEG)
    for t in range(KN):
        am = jnp.argmax(sims, axis=1)[:, None]
        idx_ref[:, t:t + 1] = am.astype(jnp.int32)
        if t + 1 < KN:
            sims = jnp.where(iota == am, NEG, sims)


def _vconv_body(near_ref, wkk_ref, bkk_ref, wk1_ref, bk1_ref, wfc_ref,
                bfc_ref, out_ref):
    w = jnp.zeros((BN, KN), jnp.float32)
    for a in range(KN):
        na = near_ref[a]
        mult = lax.dot_general(na, wkk_ref[a], (((1,), (1,)), ((), ())),
                               preferred_element_type=jnp.float32,
                               precision=lax.Precision.HIGHEST)
        mult = mult + bkk_ref[a]
        m = jnp.max(mult, axis=1, keepdims=True)
        e = jnp.exp(mult - m)
        soft = e / jnp.sum(e, axis=1, keepdims=True)
        w = w + wk1_ref[:, a:a + 1] * soft
    pooled = w[:, 0:1] * near_ref[0]
    for b in range(1, KN):
        pooled = pooled + w[:, b:b + 1] * near_ref[b]
    pooled = pooled + bk1_ref[:, 0:1]
    out = lax.dot_general(pooled, wfc_ref[...], (((1,), (1,)), ((), ())),
                          preferred_element_type=jnp.float32,
                          precision=lax.Precision.HIGHEST)
    out_ref[...] = out + bfc_ref[...]


def _make_gather():
    mesh = plsc.VectorSubcoreMesh(core_axis_name="c", subcore_axis_name="s")
    nch = PER_W // GCHUNK

    @functools.partial(
        pl.kernel, mesh=mesh,
        out_type=jax.ShapeDtypeStruct((KN * NP, D), jnp.float32),
        scratch_types=[
            pltpu.VMEM((nch, GCHUNK), jnp.int32),
            pltpu.VMEM((2, GCHUNK, D), jnp.float32),
            pltpu.SemaphoreType.DMA,
            pltpu.SemaphoreType.DMA,
        ],
    )
    def gk(idx_hbm, table_hbm, out_hbm, idx_v, rows_v, gsem, osem):
        wid = lax.axis_index("s") * 2 + lax.axis_index("c")
        base = wid * PER_W
        pltpu.sync_copy(idx_hbm.at[wid], idx_v)

        def gstart(c, b):
            pltpu.async_copy(table_hbm.at[idx_v.at[c]], rows_v.at[b], gsem)

        def ostart(c, b):
            pltpu.async_copy(rows_v.at[b],
                             out_hbm.at[pl.ds(base + c * GCHUNK, GCHUNK)],
                             osem)

        gstart(0, 0)
        gstart(1, 1)

        def body(i, carry):
            c = 2 * i
            for b in range(2):
                pltpu.make_async_copy(table_hbm.at[idx_v.at[c + b]],
                                      rows_v.at[b], gsem).wait()
                ostart(c + b, b)
                pltpu.make_async_copy(rows_v.at[b], out_hbm.at[
                    pl.ds(base + (c + b) * GCHUNK, GCHUNK)], osem).wait()
                nxt = c + b + 2

                @pl.when(nxt < nch)
                def _():
                    gstart(nxt, b)
            return carry

        lax.fori_loop(0, nch // 2, body, 0)

    return gk


def kernel(feats, edge_dict, ite, Wkk, bkk, Wk1, bk1, W1, b1, W2, b2, Wfc,
           bfc):
    feats_p = jnp.pad(feats, ((0, NP - N), (0, 0)))

    x = pl.pallas_call(
        _normalize_body,
        out_shape=jax.ShapeDtypeStruct((NP, D), jnp.float32),
    )(feats_p)

    idx = pl.pallas_call(
        _topk_body,
        grid=(NP // RB,),
        in_specs=[
            pl.BlockSpec((RB, D), lambda i: (i, 0)),
            pl.BlockSpec((NP, D), lambda i: (0, 0)),
        ],
        out_specs=pl.BlockSpec((RB, KN), lambda i: (i, 0)),
        out_shape=jax.ShapeDtypeStruct((NP, KN), jnp.int32),
    )(x, x)

    idx_t = idx.T.reshape(NW, PER_W // GCHUNK, GCHUNK)
    nearest = _make_gather()(idx_t, feats)
    near3 = nearest.reshape(KN, NP, D)

    out_p = pl.pallas_call(
        _vconv_body,
        grid=(NP // BN,),
        in_specs=[
            pl.BlockSpec((KN, BN, D), lambda i: (0, i, 0)),
            pl.BlockSpec((KN, KN, D), lambda i: (0, 0, 0)),
            pl.BlockSpec((KN, 1, KN), lambda i: (0, 0, 0)),
            pl.BlockSpec((1, KN), lambda i: (0, 0)),
            pl.BlockSpec((1, 1), lambda i: (0, 0)),
            pl.BlockSpec((D, D), lambda i: (0, 0)),
            pl.BlockSpec((1, D), lambda i: (0, 0)),
        ],
        out_specs=pl.BlockSpec((BN, D), lambda i: (i, 0)),
        out_shape=jax.ShapeDtypeStruct((NP, D), jnp.float32),
    )(near3, Wkk.reshape(KN, KN, D), bkk.reshape(KN, 1, KN),
      Wk1.reshape(1, KN), bk1.reshape(1, 1), Wfc, bfc.reshape(1, D))

    return out_p[:N]
